# Initial kernel scaffold; baseline (speedup 1.0000x reference)
import jax
import jax.numpy as jnp
from jax.experimental import pallas as pl


def kernel(logits_0, logits_1, logits_2, logits_3, att_weight):
    raise NotImplementedError("write your pallas kernel here")


# speedup vs baseline: 5.7112x; 5.7112x over previous
import jax
import jax.numpy as jnp
from jax.experimental import pallas as pl
from jax.experimental.pallas import tpu as pltpu

_TR = 512


def _att_pool_kernel(w_ref, x0_ref, x1_ref, x2_ref, x3_ref, o_ref):
    w = w_ref[...]
    x0 = x0_ref[...]
    x1 = x1_ref[...]
    x2 = x2_ref[...]
    x3 = x3_ref[...]

    s0 = jnp.sum(x0 * w, axis=1, keepdims=True)
    s1 = jnp.sum(x1 * w, axis=1, keepdims=True)
    s2 = jnp.sum(x2 * w, axis=1, keepdims=True)
    s3 = jnp.sum(x3 * w, axis=1, keepdims=True)

    m = jnp.maximum(jnp.maximum(s0, s1), jnp.maximum(s2, s3))
    e0 = jnp.exp(s0 - m)
    e1 = jnp.exp(s1 - m)
    e2 = jnp.exp(s2 - m)
    e3 = jnp.exp(s3 - m)
    inv = 1.0 / ((e0 + e1) + (e2 + e3))

    acc = (e0 * inv) * x0 + (e1 * inv) * x1
    acc = acc + (e2 * inv) * x2 + (e3 * inv) * x3
    o_ref[...] = acc


def kernel(logits_0, logits_1, logits_2, logits_3, att_weight):
    B, D = logits_0.shape
    TR = _TR if B % _TR == 0 else B
    grid = (B // TR,)

    w = att_weight.reshape(1, D).astype(jnp.float32)

    x_spec = pl.BlockSpec((TR, D), lambda r: (r, 0))
    w_spec = pl.BlockSpec((1, D), lambda r: (0, 0))

    out = pl.pallas_call(
        _att_pool_kernel,
        out_shape=jax.ShapeDtypeStruct((B, D), logits_0.dtype),
        grid=grid,
        in_specs=[w_spec] + [x_spec] * 4,
        out_specs=pl.BlockSpec((TR, D), lambda r: (r, 0)),
        compiler_params=pltpu.CompilerParams(
            dimension_semantics=("parallel",),
            vmem_limit_bytes=64 << 20,
        ),
    )(w, logits_0, logits_1, logits_2, logits_3)

    return jnp.squeeze(out)


# speedup vs baseline: 6.3120x; 1.1052x over previous
import jax
import jax.numpy as jnp
from jax.experimental import pallas as pl
from jax.experimental.pallas import tpu as pltpu

_TR = 1024


def _att_pool_kernel(w_ref, x0_ref, x1_ref, x2_ref, x3_ref, o_ref):
    w = w_ref[...]
    x0 = x0_ref[...]
    x1 = x1_ref[...]
    x2 = x2_ref[...]
    x3 = x3_ref[...]

    s0 = jnp.sum(x0 * w, axis=1, keepdims=True)
    s1 = jnp.sum(x1 * w, axis=1, keepdims=True)
    s2 = jnp.sum(x2 * w, axis=1, keepdims=True)
    s3 = jnp.sum(x3 * w, axis=1, keepdims=True)

    m = jnp.maximum(jnp.maximum(s0, s1), jnp.maximum(s2, s3))
    e0 = jnp.exp(s0 - m)
    e1 = jnp.exp(s1 - m)
    e2 = jnp.exp(s2 - m)
    e3 = jnp.exp(s3 - m)
    inv = 1.0 / ((e0 + e1) + (e2 + e3))

    acc = (e0 * inv) * x0 + (e1 * inv) * x1
    acc = acc + (e2 * inv) * x2 + (e3 * inv) * x3
    o_ref[...] = acc


def kernel(logits_0, logits_1, logits_2, logits_3, att_weight):
    B, D = logits_0.shape
    TR = _TR if B % _TR == 0 else B
    grid = (B // TR,)

    w = att_weight.reshape(1, D).astype(jnp.float32)

    x_spec = pl.BlockSpec((TR, D), lambda r: (r, 0))
    w_spec = pl.BlockSpec((1, D), lambda r: (0, 0))

    out = pl.pallas_call(
        _att_pool_kernel,
        out_shape=jax.ShapeDtypeStruct((B, D), logits_0.dtype),
        grid=grid,
        in_specs=[w_spec] + [x_spec] * 4,
        out_specs=pl.BlockSpec((TR, D), lambda r: (r, 0)),
        compiler_params=pltpu.CompilerParams(
            dimension_semantics=("parallel",),
            vmem_limit_bytes=64 << 20,
        ),
    )(w, logits_0, logits_1, logits_2, logits_3)

    return jnp.squeeze(out)


# speedup vs baseline: 6.4229x; 1.0176x over previous
import jax
import jax.numpy as jnp
from jax.experimental import pallas as pl
from jax.experimental.pallas import tpu as pltpu

_TR = 2048


def _att_pool_kernel(w_ref, x0_ref, x1_ref, x2_ref, x3_ref, o_ref):
    w = w_ref[...]
    x0 = x0_ref[...]
    x1 = x1_ref[...]
    x2 = x2_ref[...]
    x3 = x3_ref[...]

    s0 = jnp.sum(x0 * w, axis=1, keepdims=True)
    s1 = jnp.sum(x1 * w, axis=1, keepdims=True)
    s2 = jnp.sum(x2 * w, axis=1, keepdims=True)
    s3 = jnp.sum(x3 * w, axis=1, keepdims=True)

    m = jnp.maximum(jnp.maximum(s0, s1), jnp.maximum(s2, s3))
    e0 = jnp.exp(s0 - m)
    e1 = jnp.exp(s1 - m)
    e2 = jnp.exp(s2 - m)
    e3 = jnp.exp(s3 - m)
    inv = 1.0 / ((e0 + e1) + (e2 + e3))

    acc = (e0 * inv) * x0 + (e1 * inv) * x1
    acc = acc + (e2 * inv) * x2 + (e3 * inv) * x3
    o_ref[...] = acc


def kernel(logits_0, logits_1, logits_2, logits_3, att_weight):
    B, D = logits_0.shape
    TR = _TR if B % _TR == 0 else B
    grid = (B // TR,)

    w = att_weight.reshape(1, D).astype(jnp.float32)

    x_spec = pl.BlockSpec((TR, D), lambda r: (r, 0))
    w_spec = pl.BlockSpec((1, D), lambda r: (0, 0))

    out = pl.pallas_call(
        _att_pool_kernel,
        out_shape=jax.ShapeDtypeStruct((B, D), logits_0.dtype),
        grid=grid,
        in_specs=[w_spec] + [x_spec] * 4,
        out_specs=pl.BlockSpec((TR, D), lambda r: (r, 0)),
        compiler_params=pltpu.CompilerParams(
            dimension_semantics=("parallel",),
            vmem_limit_bytes=64 << 20,
        ),
    )(w, logits_0, logits_1, logits_2, logits_3)

    return jnp.squeeze(out)
